# Initial kernel scaffold; baseline (speedup 1.0000x reference)
import jax
import jax.numpy as jnp
from jax.experimental import pallas as pl


def kernel(x_class, x_attr, x_type, x_value, edges, W1, b1, W2, b2, fc1_w, fc1_b, bn_g, bn_b, fc2_w, fc2_b, bn2_g, bn2_b, cls_w, cls_b):
    raise NotImplementedError("write your pallas kernel here")


# speedup vs baseline: 2.7086x; 2.7086x over previous
import functools

import jax
import jax.numpy as jnp
from jax import lax
from jax.experimental import pallas as pl
from jax.experimental.pallas import tpu as pltpu
from jax.experimental.pallas import tpu_sc as plsc

N = 10000
E = 50000
D = 128
H = 128
NC = 40
R = 11
SRC_T = [1, 2, 0, 2, 3, 1, 0, 1, 3, 2, 1]

NP_ = 10240
EP = 51200
ROWS_PER_TILE = NP_ // 16
CHUNK = 128
CHUNKS_PER_TILE = EP // 16 // CHUNK
PAD_NODE = N

_mesh = plsc.VectorSubcoreMesh(core_axis_name="c", subcore_axis_name="s",
                               num_cores=2)


HALF = NP_ // 2
HROWS = HALF // 16


@functools.partial(
    pl.kernel,
    mesh=_mesh,
    out_type=jax.ShapeDtypeStruct((22 * NP_, D), jnp.float32),
    scratch_types=[
        pltpu.VMEM_SHARED((HALF + 8, D), jnp.float32),
        pltpu.VMEM((HROWS, D), jnp.float32),
        pltpu.VMEM((CHUNK, D), jnp.float32),
        pltpu.VMEM((CHUNK,), jnp.int32),
    ],
)
def _sc_degrees(e_all, zeros_r, ones_r, degs_out, acc, zbuf, ones_v, idx_v):
    c = lax.axis_index("c")
    w = lax.axis_index("s")
    pltpu.sync_copy(zeros_r, zbuf)
    pltpu.sync_copy(ones_r, ones_v)
    for slot in range(11):
        p = slot * 2 + c
        for hf in range(2):
            pltpu.sync_copy(zbuf, acc.at[pl.ds(w * HROWS, HROWS)])
            plsc.subcore_barrier()

            def chunk_body(k, _, p=p, hf=hf):
                off = p * EP + (w * CHUNKS_PER_TILE + k) * CHUNK
                off = pl.multiple_of(off, CHUNK)
                pltpu.sync_copy(e_all.at[pl.ds(off, CHUNK)], idx_v)
                for j in range(CHUNK // 16):
                    v = idx_v[pl.ds(16 * j, 16)] - hf * HALF
                    inb = (v >= 0) & (v < HALF)
                    idx_v[pl.ds(16 * j, 16)] = jnp.where(inb, v, HALF)
                pltpu.sync_copy(ones_v, acc.at[idx_v], add=True)
                return _

            lax.fori_loop(0, CHUNKS_PER_TILE, chunk_body, None)
            plsc.subcore_barrier()
            pltpu.sync_copy(
                acc.at[pl.ds(w * HROWS, HROWS)],
                degs_out.at[pl.ds(p * NP_ + hf * HALF + w * HROWS, HROWS)],
            )
            plsc.subcore_barrier()


HALF = NP_ // 2
HROWS = HALF // 16


@functools.partial(
    pl.kernel,
    mesh=_mesh,
    out_type=jax.ShapeDtypeStruct((R * NP_, D), jnp.float32),
    scratch_types=[
        pltpu.VMEM_SHARED((HALF + 8, D), jnp.float32),
        pltpu.VMEM((HROWS, D), jnp.float32),
        pltpu.VMEM((CHUNK, D), jnp.float32),
        pltpu.VMEM((CHUNK,), jnp.int32),
        pltpu.VMEM((CHUNK,), jnp.int32),
    ],
)
def _sc_scatter(e_src, e_dst, y_flat, zeros_r, m_out, acc, zbuf, rows_v,
                sidx, didx):
    c = lax.axis_index("c")
    w = lax.axis_index("s")
    pltpu.sync_copy(zeros_r, zbuf)
    for slot in range(6):
        rel = slot * 2 + c

        @pl.when(rel < R)
        def _():
            for hf in range(2):
                pltpu.sync_copy(zbuf, acc.at[pl.ds(w * HROWS, HROWS)])
                plsc.subcore_barrier()

                def chunk_body(k, _, hf=hf):
                    eoff = rel * EP + (w * CHUNKS_PER_TILE + k) * CHUNK
                    eoff = pl.multiple_of(eoff, CHUNK)
                    pltpu.sync_copy(e_src.at[pl.ds(eoff, CHUNK)], sidx)
                    pltpu.sync_copy(e_dst.at[pl.ds(eoff, CHUNK)], didx)
                    pltpu.sync_copy(y_flat.at[sidx], rows_v)
                    for j in range(CHUNK // 16):
                        v = didx[pl.ds(16 * j, 16)] - hf * HALF
                        inb = (v >= 0) & (v < HALF)
                        didx[pl.ds(16 * j, 16)] = jnp.where(inb, v, HALF)
                    pltpu.sync_copy(rows_v, acc.at[didx], add=True)
                    return _

                lax.fori_loop(0, CHUNKS_PER_TILE, chunk_body, None)
                plsc.subcore_barrier()
                pltpu.sync_copy(
                    acc.at[pl.ds(w * HROWS, HROWS)],
                    m_out.at[pl.ds(rel * NP_ + hf * HALF + w * HROWS, HROWS)],
                )
                plsc.subcore_barrier()


TYPE_RELS = [[r for r in range(R) if SRC_T[r] == t] for t in range(4)]
PERM = [r for lt in TYPE_RELS for r in lt]
POS = [PERM.index(r) for r in range(R)]


def _mm_body(x_ref, deg_ref, w_ref, y_ref):
    x = x_ref[0]
    wm = w_ref[0]
    s = lax.rsqrt(jnp.clip(deg_ref[0, :, 0], 1.0, None))
    y_ref[0] = jnp.dot(x, wm, preferred_element_type=jnp.float32) * s[:, None]


def _relmm(feats, degs, W):
    din = feats.shape[-1]
    outs = []
    for t in range(4):
        lt = TYPE_RELS[t]
        k = len(lt)
        degs_t = degs[jnp.array([2 * r for r in lt])]
        w_t = W[jnp.array(lt)]
        outs.append(pl.pallas_call(
            _mm_body,
            grid=(k, NP_ // ROWS_PER_TILE),
            in_specs=[
                pl.BlockSpec((1, ROWS_PER_TILE, din),
                             lambda r, i, t=t: (t, i, 0)),
                pl.BlockSpec((1, ROWS_PER_TILE, 16), lambda r, i: (r, i, 0)),
                pl.BlockSpec((1, din, H), lambda r, i: (r, 0, 0)),
            ],
            out_specs=pl.BlockSpec((1, ROWS_PER_TILE, H),
                                   lambda r, i: (r, i, 0)),
            out_shape=jax.ShapeDtypeStruct((k, NP_, H), jnp.float32),
        )(feats, degs_t, w_t))
    return jnp.concatenate(outs, axis=0)


def _combine_body(mask_pad, m_ref, deg_ref, b_ref, h_ref):
    sc = []
    for r in range(R):
        s = lax.rsqrt(jnp.clip(deg_ref[2 * r + 1, :, 0], 1.0, None))
        sc.append(m_ref[r] * s[:, None] + b_ref[r][None, :])
    z = jnp.zeros_like(sc[0])
    h_class = jnp.concatenate([sc[0], z, sc[1]], axis=1)
    h_attr = jnp.concatenate(
        [z, jnp.maximum(jnp.maximum(sc[2], sc[3]), sc[4]), sc[5]], axis=1)
    h_type = jnp.concatenate(
        [z, jnp.maximum(jnp.maximum(sc[6], sc[7]), sc[8]), sc[9]], axis=1)
    h_value = jnp.concatenate([z, z, sc[10]], axis=1)
    h = jnp.stack([h_class, h_attr, h_type, h_value])
    h = jnp.maximum(h, 0.0)
    if mask_pad:
        i = pl.program_id(0)
        gid = i * ROWS_PER_TILE + lax.broadcasted_iota(jnp.int32, h.shape, 1)
        h = jnp.where(gid < N, h, 0.0)
    h_ref[...] = h


def _combine(M, degs, b, mask_pad):
    return pl.pallas_call(
        functools.partial(_combine_body, mask_pad),
        grid=(NP_ // ROWS_PER_TILE,),
        in_specs=[
            pl.BlockSpec((R, ROWS_PER_TILE, H), lambda i: (0, i, 0)),
            pl.BlockSpec((22, ROWS_PER_TILE, 16), lambda i: (0, i, 0)),
            pl.BlockSpec((R, H), lambda i: (0, 0)),
        ],
        out_specs=pl.BlockSpec((4, ROWS_PER_TILE, 3 * H), lambda i: (0, i, 0)),
        out_shape=jax.ShapeDtypeStruct((4, NP_, 3 * H), jnp.float32),
    )(M, degs, b)


def _head_body(h_ref, fc1w, fc1b, bng, bnb, fc2w, fc2b, bn2g, bn2b, clsw, clsb,
               out_ref, acc):
    i = pl.program_id(0)
    bm = jnp.max(h_ref[...], axis=1)

    @pl.when(i == 0)
    def _():
        acc[...] = bm

    @pl.when(i > 0)
    def _():
        acc[...] = jnp.maximum(acc[...], bm)

    @pl.when(i == pl.num_programs(0) - 1)
    def _():
        bninv = 1.0 / jnp.sqrt(1.0 + 1e-5)
        hg = acc[...].reshape(1, 12 * H)
        x = jnp.dot(hg, fc1w[...], preferred_element_type=jnp.float32) + fc1b[...]
        x = x * bninv * bng[...] + bnb[...]
        x = jnp.maximum(x, 0.0)
        x = jnp.dot(x, fc2w[...], preferred_element_type=jnp.float32) + fc2b[...]
        x = x * bninv * bn2g[...] + bn2b[...]
        x = jnp.maximum(x, 0.0)
        out_ref[...] = (
            jnp.dot(x, clsw[...], preferred_element_type=jnp.float32) + clsb[...])


def _head(h2, fc1_w, fc1_b, bn_g, bn_b, fc2_w, fc2_b, bn2_g, bn2_b, cls_w, cls_b):
    full = lambda *s: pl.BlockSpec(s, lambda i: tuple(0 for _ in s))
    return pl.pallas_call(
        _head_body,
        grid=(NP_ // ROWS_PER_TILE,),
        in_specs=[
            pl.BlockSpec((4, ROWS_PER_TILE, 3 * H), lambda i: (0, i, 0)),
            full(12 * H, 3 * H), full(3 * H), full(3 * H), full(3 * H),
            full(3 * H, H), full(H), full(H), full(H),
            full(H, NC), full(NC),
        ],
        out_specs=pl.BlockSpec((1, NC), lambda i: (0, 0)),
        out_shape=jax.ShapeDtypeStruct((1, NC), jnp.float32),
        scratch_shapes=[pltpu.VMEM((4, 3 * H), jnp.float32)],
    )(h2, fc1_w, fc1_b, bn_g, bn_b, fc2_w, fc2_b, bn2_g, bn2_b, cls_w, cls_b)


def kernel(x_class, x_attr, x_type, x_value, edges, W1, b1, W2, b2,
           fc1_w, fc1_b, bn_g, bn_b, fc2_w, fc2_b, bn2_g, bn2_b, cls_w, cls_b):
    feats = jnp.stack([x_class, x_attr, x_type, x_value])
    feats = jnp.pad(feats, ((0, 0), (0, NP_ - N), (0, 0)))

    e_all = edges.reshape(2 * R, E)
    e_all = jnp.pad(e_all, ((0, 0), (0, EP - E)), constant_values=PAD_NODE)
    e_all_flat = e_all.reshape(-1)
    e_src = (e_all[0::2] + (jnp.array(POS, dtype=jnp.int32) * NP_)[:, None]
             ).reshape(-1)
    e_dst = e_all[1::2].reshape(-1)

    zeros_r = jnp.zeros((HROWS, D), jnp.float32)
    ones_r = jnp.ones((CHUNK, D), jnp.float32)

    degs = _sc_degrees(e_all_flat, zeros_r, ones_r)
    degs = degs.reshape(22, NP_, D)[:, :, :16]

    y1 = _relmm(feats, degs, W1).reshape(R * NP_, H)
    M1 = _sc_scatter(e_src, e_dst, y1, zeros_r).reshape(R, NP_, H)
    h1 = _combine(M1, degs, b1, mask_pad=False)

    y2 = _relmm(h1, degs, W2).reshape(R * NP_, H)
    M2 = _sc_scatter(e_src, e_dst, y2, zeros_r).reshape(R, NP_, H)
    h2 = _combine(M2, degs, b2, mask_pad=True)

    return _head(h2, fc1_w, fc1_b, bn_g, bn_b, fc2_w, fc2_b, bn2_g, bn2_b,
                 cls_w, cls_b)


# speedup vs baseline: 3.2359x; 1.1947x over previous
import functools

import jax
import jax.numpy as jnp
from jax import lax
from jax.experimental import pallas as pl
from jax.experimental.pallas import tpu as pltpu
from jax.experimental.pallas import tpu_sc as plsc

N = 10000
E = 50000
D = 128
H = 128
NC = 40
R = 11
SRC_T = [1, 2, 0, 2, 3, 1, 0, 1, 3, 2, 1]

NP_ = 10240
EP = 51200
ROWS_PER_TILE = NP_ // 16
CHUNK = 128
CHUNKS_PER_TILE = EP // 16 // CHUNK
PAD_NODE = N

_mesh = plsc.VectorSubcoreMesh(core_axis_name="c", subcore_axis_name="s",
                               num_cores=2)


HALF = NP_ // 2
HROWS = HALF // 16


@functools.partial(
    pl.kernel,
    mesh=_mesh,
    out_type=jax.ShapeDtypeStruct((22 * NP_, D), jnp.float32),
    scratch_types=[
        pltpu.VMEM_SHARED((HALF + 8, D), jnp.float32),
        pltpu.VMEM((HROWS, D), jnp.float32),
        pltpu.VMEM((CHUNK, D), jnp.float32),
        pltpu.VMEM((CHUNK,), jnp.int32),
    ],
)
def _sc_degrees(e_all, zeros_r, ones_r, degs_out, acc, zbuf, ones_v, idx_v):
    c = lax.axis_index("c")
    w = lax.axis_index("s")
    pltpu.sync_copy(zeros_r, zbuf)
    pltpu.sync_copy(ones_r, ones_v)
    for slot in range(11):
        p = slot * 2 + c
        for hf in range(2):
            pltpu.sync_copy(zbuf, acc.at[pl.ds(w * HROWS, HROWS)])
            plsc.subcore_barrier()

            def chunk_body(k, _, p=p, hf=hf):
                off = p * EP + (w * CHUNKS_PER_TILE + k) * CHUNK
                off = pl.multiple_of(off, CHUNK)
                pltpu.sync_copy(e_all.at[pl.ds(off, CHUNK)], idx_v)
                for j in range(CHUNK // 16):
                    v = idx_v[pl.ds(16 * j, 16)] - hf * HALF
                    inb = (v >= 0) & (v < HALF)
                    idx_v[pl.ds(16 * j, 16)] = jnp.where(inb, v, HALF)
                pltpu.sync_copy(ones_v, acc.at[idx_v], add=True)
                return _

            lax.fori_loop(0, CHUNKS_PER_TILE, chunk_body, None)
            plsc.subcore_barrier()
            pltpu.sync_copy(
                acc.at[pl.ds(w * HROWS, HROWS)],
                degs_out.at[pl.ds(p * NP_ + hf * HALF + w * HROWS, HROWS)],
            )
            plsc.subcore_barrier()


HALF = NP_ // 2
HROWS = HALF // 16


@functools.partial(
    pl.kernel,
    mesh=_mesh,
    out_type=jax.ShapeDtypeStruct((R * NP_, D), jnp.float32),
    scratch_types=[
        pltpu.VMEM_SHARED((HALF + 8, D), jnp.float32),
        pltpu.VMEM((HROWS, D), jnp.float32),
        pltpu.VMEM((2, CHUNK, D), jnp.float32),
        pltpu.VMEM((2, CHUNK), jnp.int32),
        pltpu.VMEM((2, CHUNK), jnp.int32),
        pltpu.SemaphoreType.DMA,
    ],
)
def _sc_scatter(e_src, e_dst, y_flat, zeros_r, m_out, acc, zbuf, rows2,
                sidx2, didx2, sem):
    c = lax.axis_index("c")
    w = lax.axis_index("s")
    pltpu.sync_copy(zeros_r, zbuf)
    for slot in range(6):
        rel = slot * 2 + c

        @pl.when(rel < R)
        def _():
            def load_issue(k, b):
                eoff = rel * EP + (w * CHUNKS_PER_TILE + k) * CHUNK
                eoff = pl.multiple_of(eoff, CHUNK)
                pltpu.sync_copy(e_src.at[pl.ds(eoff, CHUNK)], sidx2.at[b])
                pltpu.sync_copy(e_dst.at[pl.ds(eoff, CHUNK)], didx2.at[b])
                pltpu.async_copy(y_flat.at[sidx2.at[b]], rows2.at[b], sem)

            def process(b, hf):
                pltpu.make_async_copy(
                    y_flat.at[sidx2.at[b]], rows2.at[b], sem).wait()
                for j in range(CHUNK // 16):
                    v = didx2[b, pl.ds(16 * j, 16)] - hf * HALF
                    inb = (v >= 0) & (v < HALF)
                    didx2[b, pl.ds(16 * j, 16)] = jnp.where(inb, v, HALF)
                pltpu.sync_copy(rows2.at[b], acc.at[didx2.at[b]], add=True)

            for hf in range(2):
                pltpu.sync_copy(zbuf, acc.at[pl.ds(w * HROWS, HROWS)])
                plsc.subcore_barrier()
                load_issue(0, 0)

                def pair_body(m, _, hf=hf):
                    a = 2 * m
                    load_issue(a + 1, 1)
                    process(0, hf)
                    load_issue(a + 2, 0)
                    process(1, hf)
                    return _

                lax.fori_loop(0, CHUNKS_PER_TILE // 2, pair_body, None)
                process(0, hf)
                plsc.subcore_barrier()
                pltpu.sync_copy(
                    acc.at[pl.ds(w * HROWS, HROWS)],
                    m_out.at[pl.ds(rel * NP_ + hf * HALF + w * HROWS, HROWS)],
                )
                plsc.subcore_barrier()


TYPE_RELS = [[r for r in range(R) if SRC_T[r] == t] for t in range(4)]
PERM = [r for lt in TYPE_RELS for r in lt]
POS = [PERM.index(r) for r in range(R)]


def _mm_body(x_ref, deg_ref, w_ref, y_ref):
    x = x_ref[0]
    wm = w_ref[0]
    s = lax.rsqrt(jnp.clip(deg_ref[0, :, 0], 1.0, None))
    y_ref[0] = jnp.dot(x, wm, preferred_element_type=jnp.float32) * s[:, None]


def _relmm(feats, degs, W):
    din = feats.shape[-1]
    outs = []
    for t in range(4):
        lt = TYPE_RELS[t]
        k = len(lt)
        degs_t = degs[jnp.array([2 * r for r in lt])]
        w_t = W[jnp.array(lt)]
        outs.append(pl.pallas_call(
            _mm_body,
            grid=(k, NP_ // ROWS_PER_TILE),
            in_specs=[
                pl.BlockSpec((1, ROWS_PER_TILE, din),
                             lambda r, i, t=t: (t, i, 0)),
                pl.BlockSpec((1, ROWS_PER_TILE, 16), lambda r, i: (r, i, 0)),
                pl.BlockSpec((1, din, H), lambda r, i: (r, 0, 0)),
            ],
            out_specs=pl.BlockSpec((1, ROWS_PER_TILE, H),
                                   lambda r, i: (r, i, 0)),
            out_shape=jax.ShapeDtypeStruct((k, NP_, H), jnp.float32),
        )(feats, degs_t, w_t))
    return jnp.concatenate(outs, axis=0)


def _combine_body(mask_pad, m_ref, deg_ref, b_ref, h_ref):
    sc = []
    for r in range(R):
        s = lax.rsqrt(jnp.clip(deg_ref[2 * r + 1, :, 0], 1.0, None))
        sc.append(m_ref[r] * s[:, None] + b_ref[r][None, :])
    z = jnp.zeros_like(sc[0])
    h_class = jnp.concatenate([sc[0], z, sc[1]], axis=1)
    h_attr = jnp.concatenate(
        [z, jnp.maximum(jnp.maximum(sc[2], sc[3]), sc[4]), sc[5]], axis=1)
    h_type = jnp.concatenate(
        [z, jnp.maximum(jnp.maximum(sc[6], sc[7]), sc[8]), sc[9]], axis=1)
    h_value = jnp.concatenate([z, z, sc[10]], axis=1)
    h = jnp.stack([h_class, h_attr, h_type, h_value])
    h = jnp.maximum(h, 0.0)
    if mask_pad:
        i = pl.program_id(0)
        gid = i * ROWS_PER_TILE + lax.broadcasted_iota(jnp.int32, h.shape, 1)
        h = jnp.where(gid < N, h, 0.0)
    h_ref[...] = h


def _combine(M, degs, b, mask_pad):
    return pl.pallas_call(
        functools.partial(_combine_body, mask_pad),
        grid=(NP_ // ROWS_PER_TILE,),
        in_specs=[
            pl.BlockSpec((R, ROWS_PER_TILE, H), lambda i: (0, i, 0)),
            pl.BlockSpec((22, ROWS_PER_TILE, 16), lambda i: (0, i, 0)),
            pl.BlockSpec((R, H), lambda i: (0, 0)),
        ],
        out_specs=pl.BlockSpec((4, ROWS_PER_TILE, 3 * H), lambda i: (0, i, 0)),
        out_shape=jax.ShapeDtypeStruct((4, NP_, 3 * H), jnp.float32),
    )(M, degs, b)


def _head_body(h_ref, fc1w, fc1b, bng, bnb, fc2w, fc2b, bn2g, bn2b, clsw, clsb,
               out_ref, acc):
    i = pl.program_id(0)
    bm = jnp.max(h_ref[...], axis=1)

    @pl.when(i == 0)
    def _():
        acc[...] = bm

    @pl.when(i > 0)
    def _():
        acc[...] = jnp.maximum(acc[...], bm)

    @pl.when(i == pl.num_programs(0) - 1)
    def _():
        bninv = 1.0 / jnp.sqrt(1.0 + 1e-5)
        hg = acc[...].reshape(1, 12 * H)
        x = jnp.dot(hg, fc1w[...], preferred_element_type=jnp.float32) + fc1b[...]
        x = x * bninv * bng[...] + bnb[...]
        x = jnp.maximum(x, 0.0)
        x = jnp.dot(x, fc2w[...], preferred_element_type=jnp.float32) + fc2b[...]
        x = x * bninv * bn2g[...] + bn2b[...]
        x = jnp.maximum(x, 0.0)
        out_ref[...] = (
            jnp.dot(x, clsw[...], preferred_element_type=jnp.float32) + clsb[...])


def _head(h2, fc1_w, fc1_b, bn_g, bn_b, fc2_w, fc2_b, bn2_g, bn2_b, cls_w, cls_b):
    full = lambda *s: pl.BlockSpec(s, lambda i: tuple(0 for _ in s))
    return pl.pallas_call(
        _head_body,
        grid=(NP_ // ROWS_PER_TILE,),
        in_specs=[
            pl.BlockSpec((4, ROWS_PER_TILE, 3 * H), lambda i: (0, i, 0)),
            full(12 * H, 3 * H), full(3 * H), full(3 * H), full(3 * H),
            full(3 * H, H), full(H), full(H), full(H),
            full(H, NC), full(NC),
        ],
        out_specs=pl.BlockSpec((1, NC), lambda i: (0, 0)),
        out_shape=jax.ShapeDtypeStruct((1, NC), jnp.float32),
        scratch_shapes=[pltpu.VMEM((4, 3 * H), jnp.float32)],
    )(h2, fc1_w, fc1_b, bn_g, bn_b, fc2_w, fc2_b, bn2_g, bn2_b, cls_w, cls_b)


def kernel(x_class, x_attr, x_type, x_value, edges, W1, b1, W2, b2,
           fc1_w, fc1_b, bn_g, bn_b, fc2_w, fc2_b, bn2_g, bn2_b, cls_w, cls_b):
    feats = jnp.stack([x_class, x_attr, x_type, x_value])
    feats = jnp.pad(feats, ((0, 0), (0, NP_ - N), (0, 0)))

    e_all = edges.reshape(2 * R, E)
    e_all = jnp.pad(e_all, ((0, 0), (0, EP - E)), constant_values=PAD_NODE)
    e_all_flat = e_all.reshape(-1)
    e_src = (e_all[0::2] + (jnp.array(POS, dtype=jnp.int32) * NP_)[:, None]
             ).reshape(-1)
    e_dst = e_all[1::2].reshape(-1)

    zeros_r = jnp.zeros((HROWS, D), jnp.float32)
    ones_r = jnp.ones((CHUNK, D), jnp.float32)

    degs = _sc_degrees(e_all_flat, zeros_r, ones_r)
    degs = degs.reshape(22, NP_, D)[:, :, :16]

    y1 = _relmm(feats, degs, W1).reshape(R * NP_, H)
    M1 = _sc_scatter(e_src, e_dst, y1, zeros_r).reshape(R, NP_, H)
    h1 = _combine(M1, degs, b1, mask_pad=False)

    y2 = _relmm(h1, degs, W2).reshape(R * NP_, H)
    M2 = _sc_scatter(e_src, e_dst, y2, zeros_r).reshape(R, NP_, H)
    h2 = _combine(M2, degs, b2, mask_pad=True)

    return _head(h2, fc1_w, fc1_b, bn_g, bn_b, fc2_w, fc2_b, bn2_g, bn2_b,
                 cls_w, cls_b)
